# baseline (speedup 1.0000x reference)
import jax
import jax.numpy as jnp
from jax import lax
from jax.experimental import pallas as pl
from jax.experimental.pallas import tpu as pltpu
from jax.experimental.pallas import tpu_sc as plsc

N_NODES = 10000
D_IN = 128
D_REL = 64
WIDTH = 80
N_EDGES = 320000
CHUNK = 128
NCH = 160
NROWS = 16 * NCH
E_PAD = NROWS * CHUNK
TRASH = N_NODES
ACC_ROWS = 10112
TBL_ROWS = 10008
RPT = ACC_ROWS // 16
IBC = 20


def _proj_body(xu_ref, wu_ref, xi_ref, wi_ref, au_ref, ai_ref):
    ones_then_zeros = (
        lax.broadcasted_iota(jnp.int32, (N_NODES, WIDTH - D_REL), 1) == 0
    ).astype(jnp.float32)
    for x_ref, w_ref, o_ref in ((xu_ref, wu_ref, au_ref), (xi_ref, wi_ref, ai_ref)):
        xw = jnp.dot(x_ref[...], w_ref[...], preferred_element_type=jnp.float32)
        o_ref[0:N_NODES, :] = jnp.concatenate([xw, ones_then_zeros], axis=1)
        o_ref[N_NODES:TBL_ROWS, :] = jnp.zeros(
            (TBL_ROWS - N_NODES, WIDTH), jnp.float32)


def _proj_xp_body(xu_ref, wu_ref, xi_ref, wi_ref, au_ref, ai_ref,
                  pu_ref, pi_ref):
    _proj_body(xu_ref, wu_ref, xi_ref, wi_ref, au_ref, ai_ref)
    pu_ref[...] = au_ref[0:N_NODES, 0:D_REL]
    pi_ref[...] = ai_ref[0:N_NODES, 0:D_REL]


def _project(x_user, W_user, x_item, W_item):
    out = jax.ShapeDtypeStruct((TBL_ROWS, WIDTH), jnp.float32)
    xp = jax.ShapeDtypeStruct((N_NODES, D_REL), jnp.float32)
    return pl.pallas_call(
        _proj_xp_body,
        out_shape=(out, out, xp, xp),
    )(x_user, W_user, x_item, W_item)


NBUF = 5
PREF = NBUF - 2


def _edge_body(au_hbm, ai_hbm, eidx_hbm, z_hbm, out_hbm,
               gidx, sidx, r0b, r1b, r2b, r3b, r4b, cbuf, acc,
               g0, g1, g2, g3, g4, s0, s1, s2, s3, s4):
    c = lax.axis_index("c")
    s = lax.axis_index("s")
    R = (r0b, r1b, r2b, r3b, r4b)
    G = (g0, g1, g2, g3, g4)
    S = (s0, s1, s2, s3, s4)

    pltpu.sync_copy(z_hbm, r0b)
    base = s * RPT
    for k in range(5):
        n = CHUNK if k < 4 else RPT - 4 * CHUNK
        pltpu.sync_copy(r0b.at[pl.ds(0, n)], acc.at[pl.ds(base + k * CHUNK, n)])
    plsc.subcore_barrier()

    def wg(b):
        pltpu.make_async_copy(z_hbm, R[b], G[b]).wait()

    def ws(b):
        pltpu.make_async_copy(z_hbm, R[b], S[b]).wait()

    def main_loop(table):
        start = s * NCH

        def gat(j, b):
            pltpu.async_copy(table.at[gidx.at[j]], R[b], G[b])

        def sca(j, b):
            pltpu.async_copy(R[b], acc.at[sidx.at[j]], S[b], add=True)

        @pl.loop(0, NCH // IBC)
        def _(blk):
            pltpu.sync_copy(eidx_hbm.at[c, pl.ds(start + blk * IBC, IBC)], gidx)
            pltpu.sync_copy(eidx_hbm.at[1 - c, pl.ds(start + blk * IBC, IBC)], sidx)
            for t in range(PREF):
                gat(t, t)

            @pl.loop(0, IBC, step=NBUF)
            def _(j):
                for b in range(NBUF):
                    jj = j + b
                    wg(b)
                    sca(jj, b)
                    b2 = (b + PREF) % NBUF

                    @pl.when(jj + PREF < IBC)
                    def _():
                        @pl.when(jj >= NBUF - PREF)
                        def _():
                            ws(b2)
                        gat(jj + PREF, b2)

            for b in range(NBUF):
                ws(b)

    @pl.when(c == 0)
    def _():
        main_loop(au_hbm)

    @pl.when(c == 1)
    def _():
        main_loop(ai_hbm)

    plsc.subcore_barrier()

    for k in range(5):
        n = CHUNK if k < 4 else RPT - 4 * CHUNK
        r0 = base + k * CHUNK
        pltpu.sync_copy(acc.at[pl.ds(r0, n)], r0b.at[pl.ds(0, n)])

        @pl.loop(0, n)
        def _(r):
            degv = r0b[r, pl.ds(D_REL, 16)]
            recv = 1.0 / jnp.maximum(degv, 1.0)
            rec = jnp.take(recv, jnp.zeros((16,), jnp.int32))
            for q in range(D_REL // 16):
                cbuf[r, pl.ds(q * 16, 16)] = r0b[r, pl.ds(q * 16, 16)] * rec

        pltpu.sync_copy(cbuf.at[pl.ds(0, n)], out_hbm.at[c, pl.ds(r0, n)])


def _edge_aggregate(aug_u, aug_i, eidx, zblk):
    mesh = plsc.VectorSubcoreMesh(core_axis_name="c", subcore_axis_name="s")
    run = pl.kernel(
        _edge_body,
        out_type=jax.ShapeDtypeStruct((2, ACC_ROWS, D_REL), jnp.float32),
        mesh=mesh,
        compiler_params=pltpu.CompilerParams(use_tc_tiling_on_sc=False),
        scratch_types=[
            pltpu.VMEM((IBC, CHUNK), jnp.int32),
            pltpu.VMEM((IBC, CHUNK), jnp.int32),
            pltpu.VMEM((CHUNK, WIDTH), jnp.float32),
            pltpu.VMEM((CHUNK, WIDTH), jnp.float32),
            pltpu.VMEM((CHUNK, WIDTH), jnp.float32),
            pltpu.VMEM((CHUNK, WIDTH), jnp.float32),
            pltpu.VMEM((CHUNK, WIDTH), jnp.float32),
            pltpu.VMEM((CHUNK, D_REL), jnp.float32),
            pltpu.VMEM_SHARED((ACC_ROWS, WIDTH), jnp.float32),
        ] + [pltpu.SemaphoreType.DMA] * 10,
    )
    return run(aug_u, aug_i, eidx, zblk)


def kernel(x_user, x_item, edge_index, W_user, W_item):
    aug_u, aug_i, xp_u, xp_i = _project(x_user, W_user, x_item, W_item)

    eidx = jnp.concatenate([
        edge_index.astype(jnp.int32),
        jnp.full((2, E_PAD - N_EDGES), TRASH, jnp.int32),
    ], axis=1).reshape(2, NROWS, CHUNK)
    zblk = jnp.zeros((CHUNK, WIDTH), jnp.float32)

    normed = _edge_aggregate(aug_u, aug_i, eidx, zblk)
    return (normed[0, :N_NODES], xp_u, normed[1, :N_NODES], xp_i)


# speedup vs baseline: 1.2502x; 1.2502x over previous
import jax
import jax.numpy as jnp
from jax import lax
from jax.experimental import pallas as pl
from jax.experimental.pallas import tpu as pltpu
from jax.experimental.pallas import tpu_sc as plsc

N_NODES = 10000
D_IN = 128
D_REL = 64
WIDTH = 80
N_EDGES = 320000
CHUNK = 128
NCH = 160
NROWS = 16 * NCH
E_PAD = NROWS * CHUNK
TRASH = N_NODES
ACC_ROWS = 10112
TBL_ROWS = 10008
RPT = ACC_ROWS // 16
IBC = 20


def _proj_body(xu_ref, wu_ref, xi_ref, wi_ref, au_ref, ai_ref):
    ones_then_zeros = (
        lax.broadcasted_iota(jnp.int32, (N_NODES, WIDTH - D_REL), 1) == 0
    ).astype(jnp.float32)
    for x_ref, w_ref, o_ref in ((xu_ref, wu_ref, au_ref), (xi_ref, wi_ref, ai_ref)):
        xw = jnp.dot(x_ref[...], w_ref[...], preferred_element_type=jnp.float32)
        o_ref[0:N_NODES, :] = jnp.concatenate([xw, ones_then_zeros], axis=1)
        o_ref[N_NODES:TBL_ROWS, :] = jnp.zeros(
            (TBL_ROWS - N_NODES, WIDTH), jnp.float32)


def _project(x_user, W_user, x_item, W_item):
    out = jax.ShapeDtypeStruct((TBL_ROWS, WIDTH), jnp.float32)
    return pl.pallas_call(
        _proj_body,
        out_shape=(out, out),
    )(x_user, W_user, x_item, W_item)


NBUF = 5
PREF = NBUF - 2


def _edge_body(au_hbm, ai_hbm, eidx_hbm, z_hbm, out_hbm,
               gidx, sidx, r0b, r1b, r2b, r3b, r4b, acc,
               g0, g1, g2, g3, g4, s0, s1, s2, s3, s4):
    c = lax.axis_index("c")
    s = lax.axis_index("s")
    R = (r0b, r1b, r2b, r3b, r4b)
    G = (g0, g1, g2, g3, g4)
    S = (s0, s1, s2, s3, s4)

    pltpu.sync_copy(z_hbm, r0b)
    base = s * RPT
    for k in range(5):
        n = CHUNK if k < 4 else RPT - 4 * CHUNK
        pltpu.sync_copy(r0b.at[pl.ds(0, n)], acc.at[pl.ds(base + k * CHUNK, n)])
    plsc.subcore_barrier()

    def wg(b):
        pltpu.make_async_copy(z_hbm, R[b], G[b]).wait()

    def ws(b):
        pltpu.make_async_copy(z_hbm, R[b], S[b]).wait()

    def main_loop(table):
        start = s * NCH

        def gat(j, b):
            pltpu.async_copy(table.at[gidx.at[j]], R[b], G[b])

        def sca(j, b):
            pltpu.async_copy(R[b], acc.at[sidx.at[j]], S[b], add=True)

        @pl.loop(0, NCH // IBC)
        def _(blk):
            pltpu.sync_copy(eidx_hbm.at[c, pl.ds(start + blk * IBC, IBC)], gidx)
            pltpu.sync_copy(eidx_hbm.at[1 - c, pl.ds(start + blk * IBC, IBC)], sidx)
            for t in range(PREF):
                gat(t, t)

            @pl.loop(0, IBC, step=NBUF)
            def _(j):
                for b in range(NBUF):
                    jj = j + b
                    wg(b)
                    sca(jj, b)
                    b2 = (b + PREF) % NBUF

                    @pl.when(jj + PREF < IBC)
                    def _():
                        @pl.when(jj >= NBUF - PREF)
                        def _():
                            ws(b2)
                        gat(jj + PREF, b2)

            for b in range(NBUF):
                ws(b)

    @pl.when(c == 0)
    def _():
        main_loop(au_hbm)

    @pl.when(c == 1)
    def _():
        main_loop(ai_hbm)

    plsc.subcore_barrier()

    for k in range(5):
        n = CHUNK if k < 4 else RPT - 4 * CHUNK
        r0 = base + k * CHUNK
        pltpu.sync_copy(acc.at[pl.ds(r0, n)], r0b.at[pl.ds(0, n)])

        @pl.loop(0, n)
        def _(r):
            degv = r0b[r, pl.ds(D_REL, 16)]
            recv = 1.0 / jnp.maximum(degv, 1.0)
            rec = jnp.take(recv, jnp.zeros((16,), jnp.int32))
            for q in range(D_REL // 16):
                r0b[r, pl.ds(q * 16, 16)] = r0b[r, pl.ds(q * 16, 16)] * rec

        pltpu.sync_copy(r0b.at[pl.ds(0, n)], out_hbm.at[c, pl.ds(r0, n)])


def _edge_aggregate(aug_u, aug_i, eidx, zblk):
    mesh = plsc.VectorSubcoreMesh(core_axis_name="c", subcore_axis_name="s")
    run = pl.kernel(
        _edge_body,
        out_type=jax.ShapeDtypeStruct((2, ACC_ROWS, WIDTH), jnp.float32),
        mesh=mesh,
        compiler_params=pltpu.CompilerParams(use_tc_tiling_on_sc=False),
        scratch_types=[
            pltpu.VMEM((IBC, CHUNK), jnp.int32),
            pltpu.VMEM((IBC, CHUNK), jnp.int32),
            pltpu.VMEM((CHUNK, WIDTH), jnp.float32),
            pltpu.VMEM((CHUNK, WIDTH), jnp.float32),
            pltpu.VMEM((CHUNK, WIDTH), jnp.float32),
            pltpu.VMEM((CHUNK, WIDTH), jnp.float32),
            pltpu.VMEM((CHUNK, WIDTH), jnp.float32),
            pltpu.VMEM_SHARED((ACC_ROWS, WIDTH), jnp.float32),
        ] + [pltpu.SemaphoreType.DMA] * 10,
    )
    return run(aug_u, aug_i, eidx, zblk)


def kernel(x_user, x_item, edge_index, W_user, W_item):
    aug_u, aug_i = _project(x_user, W_user, x_item, W_item)
    xp_u = aug_u[:N_NODES, :D_REL]
    xp_i = aug_i[:N_NODES, :D_REL]

    eidx = jnp.concatenate([
        edge_index.astype(jnp.int32),
        jnp.full((2, E_PAD - N_EDGES), TRASH, jnp.int32),
    ], axis=1).reshape(2, NROWS, CHUNK)
    zblk = jnp.zeros((CHUNK, WIDTH), jnp.float32)

    normed = _edge_aggregate(aug_u, aug_i, eidx, zblk)
    return (normed[0, :N_NODES, :D_REL], xp_u,
            normed[1, :N_NODES, :D_REL], xp_i)


# speedup vs baseline: 2.4978x; 1.9979x over previous
import jax
import jax.numpy as jnp
from jax import lax
from jax.experimental import pallas as pl
from jax.experimental.pallas import tpu as pltpu
from jax.experimental.pallas import tpu_sc as plsc

N_NODES = 10000
D_IN = 128
D_REL = 64
WIDTH = 80
N_EDGES = 320000
CHUNK = 128
NCH = 160
NROWS = N_EDGES // CHUNK
NCH_LAST = NROWS - 15 * NCH
ACC_ROWS = 10112
TBL_ROWS = 10008
RPT = ACC_ROWS // 16
IBC = 20


def _proj_body(xu_ref, wu_ref, xi_ref, wi_ref, au_ref, ai_ref):
    ones_then_zeros = (
        lax.broadcasted_iota(jnp.int32, (N_NODES, WIDTH - D_REL), 1) == 0
    ).astype(jnp.float32)
    for x_ref, w_ref, o_ref in ((xu_ref, wu_ref, au_ref), (xi_ref, wi_ref, ai_ref)):
        xw = jnp.dot(x_ref[...], w_ref[...], preferred_element_type=jnp.float32)
        o_ref[0:N_NODES, :] = jnp.concatenate([xw, ones_then_zeros], axis=1)
        o_ref[N_NODES:TBL_ROWS, :] = jnp.zeros(
            (TBL_ROWS - N_NODES, WIDTH), jnp.float32)


def _project(x_user, W_user, x_item, W_item):
    out = jax.ShapeDtypeStruct((TBL_ROWS, WIDTH), jnp.float32)
    return pl.pallas_call(
        _proj_body,
        out_shape=(out, out),
    )(x_user, W_user, x_item, W_item)


NBUF = 5
PREF = NBUF - 2


def _edge_body(au_hbm, ai_hbm, eidx_hbm, z_hbm, out_hbm,
               gidx, sidx, r0b, r1b, r2b, r3b, r4b, acc,
               g0, g1, g2, g3, g4, s0, s1, s2, s3, s4):
    c = lax.axis_index("c")
    s = lax.axis_index("s")
    R = (r0b, r1b, r2b, r3b, r4b)
    G = (g0, g1, g2, g3, g4)
    S = (s0, s1, s2, s3, s4)

    pltpu.sync_copy(z_hbm, r0b)
    base = s * RPT
    for k in range(5):
        n = CHUNK if k < 4 else RPT - 4 * CHUNK
        pltpu.sync_copy(r0b.at[pl.ds(0, n)], acc.at[pl.ds(base + k * CHUNK, n)])
    plsc.subcore_barrier()

    def wg(b):
        pltpu.make_async_copy(z_hbm, R[b], G[b]).wait()

    def ws(b):
        pltpu.make_async_copy(z_hbm, R[b], S[b]).wait()

    def main_loop(table):
        start = s * NCH

        def gat(j, b):
            pltpu.async_copy(table.at[gidx.at[j]], R[b], G[b])

        def sca(j, b):
            pltpu.async_copy(R[b], acc.at[sidx.at[j]], S[b], add=True)

        nblk = jnp.where(s == 15, NCH_LAST // IBC, NCH // IBC)

        @pl.loop(0, nblk)
        def _(blk):
            pltpu.sync_copy(eidx_hbm.at[c, pl.ds(start + blk * IBC, IBC)], gidx)
            pltpu.sync_copy(eidx_hbm.at[1 - c, pl.ds(start + blk * IBC, IBC)], sidx)
            for t in range(PREF):
                gat(t, t)

            @pl.loop(0, IBC, step=NBUF)
            def _(j):
                for b in range(NBUF):
                    jj = j + b
                    wg(b)
                    sca(jj, b)
                    b2 = (b + PREF) % NBUF

                    @pl.when(jj + PREF < IBC)
                    def _():
                        @pl.when(jj >= NBUF - PREF)
                        def _():
                            ws(b2)
                        gat(jj + PREF, b2)

            for b in range(NBUF):
                ws(b)

    @pl.when(c == 0)
    def _():
        main_loop(au_hbm)

    @pl.when(c == 1)
    def _():
        main_loop(ai_hbm)

    plsc.subcore_barrier()

    for k in range(5):
        n = CHUNK if k < 4 else RPT - 4 * CHUNK
        r0 = base + k * CHUNK
        pltpu.sync_copy(acc.at[pl.ds(r0, n)], r0b.at[pl.ds(0, n)])

        @pl.loop(0, n)
        def _(r):
            degv = r0b[r, pl.ds(D_REL, 16)]
            recv = 1.0 / jnp.maximum(degv, 1.0)
            rec = jnp.take(recv, jnp.zeros((16,), jnp.int32))
            for q in range(D_REL // 16):
                r0b[r, pl.ds(q * 16, 16)] = r0b[r, pl.ds(q * 16, 16)] * rec

        pltpu.sync_copy(r0b.at[pl.ds(0, n)], out_hbm.at[c, pl.ds(r0, n)])


def _edge_aggregate(aug_u, aug_i, eidx, zblk):
    mesh = plsc.VectorSubcoreMesh(core_axis_name="c", subcore_axis_name="s")
    run = pl.kernel(
        _edge_body,
        out_type=jax.ShapeDtypeStruct((2, ACC_ROWS, WIDTH), jnp.float32),
        mesh=mesh,
        compiler_params=pltpu.CompilerParams(use_tc_tiling_on_sc=False),
        scratch_types=[
            pltpu.VMEM((IBC, CHUNK), jnp.int32),
            pltpu.VMEM((IBC, CHUNK), jnp.int32),
            pltpu.VMEM((CHUNK, WIDTH), jnp.float32),
            pltpu.VMEM((CHUNK, WIDTH), jnp.float32),
            pltpu.VMEM((CHUNK, WIDTH), jnp.float32),
            pltpu.VMEM((CHUNK, WIDTH), jnp.float32),
            pltpu.VMEM((CHUNK, WIDTH), jnp.float32),
            pltpu.VMEM_SHARED((ACC_ROWS, WIDTH), jnp.float32),
        ] + [pltpu.SemaphoreType.DMA] * 10,
    )
    return run(aug_u, aug_i, eidx, zblk)


def kernel(x_user, x_item, edge_index, W_user, W_item):
    aug_u, aug_i = _project(x_user, W_user, x_item, W_item)
    xp_u = aug_u[:N_NODES, :D_REL]
    xp_i = aug_i[:N_NODES, :D_REL]

    eidx = edge_index.astype(jnp.int32).reshape(2, NROWS, CHUNK)
    zblk = jnp.zeros((CHUNK, WIDTH), jnp.float32)

    normed = _edge_aggregate(aug_u, aug_i, eidx, zblk)
    return (normed[0, :N_NODES, :D_REL], xp_u,
            normed[1, :N_NODES, :D_REL], xp_i)
